# Initial kernel scaffold; baseline (speedup 1.0000x reference)
import jax
import jax.numpy as jnp
from jax.experimental import pallas as pl


def kernel(input_batch, router_w, bias_router_w, weight_bank, bias_bank):
    raise NotImplementedError("write your pallas kernel here")


# speedup vs baseline: 33.0870x; 33.0870x over previous
import jax
import jax.numpy as jnp
from jax.experimental import pallas as pl
from jax.experimental.pallas import tpu as pltpu

_B, _D, _O, _E = 256, 768, 256, 16


def _topk2_coeffs(logits):
    iota = jax.lax.broadcasted_iota(jnp.int32, logits.shape, 1)
    l1 = jnp.max(logits, axis=-1, keepdims=True)
    i1 = jnp.min(jnp.where(logits == l1, iota, _E), axis=-1, keepdims=True)
    masked = jnp.where(iota == i1, -jnp.inf, logits)
    l2 = jnp.max(masked, axis=-1, keepdims=True)
    i2 = jnp.min(jnp.where(masked == l2, iota, _E), axis=-1, keepdims=True)
    p1 = 1.0 / (1.0 + jnp.exp(l2 - l1))
    return jnp.where(iota == i1, p1, 0.0) + jnp.where(iota == i2, 1.0 - p1, 0.0)


def _moe_kernel(x_ref, rw_ref, brw_ref, w_ref, bb_ref, out_ref, c_ref):
    e = pl.program_id(0)

    @pl.when(e == 0)
    def _init():
        x = x_ref[...]
        cw = _topk2_coeffs(jnp.dot(x, rw_ref[...], preferred_element_type=jnp.float32))
        cb = _topk2_coeffs(jnp.dot(x, brw_ref[...], preferred_element_type=jnp.float32))
        c_ref[...] = cw
        out_ref[...] = jnp.dot(cb, bb_ref[...], preferred_element_type=jnp.float32)

    y = jnp.dot(x_ref[...], w_ref[0], preferred_element_type=jnp.float32)
    c = c_ref[...]
    iota = jax.lax.broadcasted_iota(jnp.int32, c.shape, 1)
    ce = jnp.sum(jnp.where(iota == e, c, 0.0), axis=1, keepdims=True)
    out_ref[...] += ce * y


def kernel(input_batch, router_w, bias_router_w, weight_bank, bias_bank):
    return pl.pallas_call(
        _moe_kernel,
        grid=(_E,),
        in_specs=[
            pl.BlockSpec((_B, _D), lambda e: (0, 0)),
            pl.BlockSpec((_D, _E), lambda e: (0, 0)),
            pl.BlockSpec((_D, _E), lambda e: (0, 0)),
            pl.BlockSpec((1, _D, _O), lambda e: (e, 0, 0)),
            pl.BlockSpec((_E, _O), lambda e: (0, 0)),
        ],
        out_specs=pl.BlockSpec((_B, _O), lambda e: (0, 0)),
        out_shape=jax.ShapeDtypeStruct((_B, _O), jnp.float32),
        scratch_shapes=[pltpu.VMEM((_B, _E), jnp.float32)],
        compiler_params=pltpu.CompilerParams(
            dimension_semantics=("arbitrary",),
        ),
    )(input_batch, router_w, bias_router_w, weight_bank, bias_bank)


# speedup vs baseline: 43.6104x; 1.3181x over previous
import jax
import jax.numpy as jnp
from jax.experimental import pallas as pl
from jax.experimental.pallas import tpu as pltpu

_B, _D, _O, _E = 256, 768, 256, 16


def _topk2_coeffs(logits):
    iota = jax.lax.broadcasted_iota(jnp.int32, logits.shape, 1)
    l1 = jnp.max(logits, axis=-1, keepdims=True)
    i1 = jnp.min(jnp.where(logits == l1, iota, _E), axis=-1, keepdims=True)
    masked = jnp.where(iota == i1, -jnp.inf, logits)
    l2 = jnp.max(masked, axis=-1, keepdims=True)
    i2 = jnp.min(jnp.where(masked == l2, iota, _E), axis=-1, keepdims=True)
    p1 = 1.0 / (1.0 + jnp.exp(l2 - l1))
    return jnp.where(iota == i1, p1, 0.0) + jnp.where(iota == i2, 1.0 - p1, 0.0)


_EB = 2


def _moe_kernel(x_ref, rw_ref, brw_ref, w_ref, bb_ref, out_ref, c_ref):
    g = pl.program_id(0)

    @pl.when(g == 0)
    def _init():
        x = x_ref[...]
        cw = _topk2_coeffs(jnp.dot(x, rw_ref[...], preferred_element_type=jnp.float32))
        cb = _topk2_coeffs(jnp.dot(x, brw_ref[...], preferred_element_type=jnp.float32))
        c_ref[...] = cw
        out_ref[...] = jnp.dot(cb, bb_ref[...], preferred_element_type=jnp.float32)

    c = c_ref[...]
    iota = jax.lax.broadcasted_iota(jnp.int32, c.shape, 1)
    acc = out_ref[...]
    for j in range(_EB):
        e = g * _EB + j
        y = jnp.dot(x_ref[...], w_ref[j], preferred_element_type=jnp.float32)
        ce = jnp.sum(jnp.where(iota == e, c, 0.0), axis=1, keepdims=True)
        acc = acc + ce * y
    out_ref[...] = acc


def kernel(input_batch, router_w, bias_router_w, weight_bank, bias_bank):
    return pl.pallas_call(
        _moe_kernel,
        grid=(_E // _EB,),
        in_specs=[
            pl.BlockSpec((_B, _D), lambda e: (0, 0)),
            pl.BlockSpec((_D, _E), lambda e: (0, 0)),
            pl.BlockSpec((_D, _E), lambda e: (0, 0)),
            pl.BlockSpec((_EB, _D, _O), lambda e: (e, 0, 0)),
            pl.BlockSpec((_E, _O), lambda e: (0, 0)),
        ],
        out_specs=pl.BlockSpec((_B, _O), lambda e: (0, 0)),
        out_shape=jax.ShapeDtypeStruct((_B, _O), jnp.float32),
        scratch_shapes=[pltpu.VMEM((_B, _E), jnp.float32)],
        compiler_params=pltpu.CompilerParams(
            dimension_semantics=("arbitrary",),
        ),
    )(input_batch, router_w, bias_router_w, weight_bank, bias_bank)


# speedup vs baseline: 51.5478x; 1.1820x over previous
import jax
import jax.numpy as jnp
from jax.experimental import pallas as pl
from jax.experimental.pallas import tpu as pltpu

_B, _D, _O, _E = 256, 768, 256, 16


def _topk2_coeffs(logits):
    iota = jax.lax.broadcasted_iota(jnp.int32, logits.shape, 1)
    l1 = jnp.max(logits, axis=-1, keepdims=True)
    i1 = jnp.min(jnp.where(logits == l1, iota, _E), axis=-1, keepdims=True)
    masked = jnp.where(iota == i1, -jnp.inf, logits)
    l2 = jnp.max(masked, axis=-1, keepdims=True)
    i2 = jnp.min(jnp.where(masked == l2, iota, _E), axis=-1, keepdims=True)
    p1 = 1.0 / (1.0 + jnp.exp(l2 - l1))
    return jnp.where(iota == i1, p1, 0.0) + jnp.where(iota == i2, 1.0 - p1, 0.0)


_EB = 4


def _moe_kernel(x_ref, rw_ref, brw_ref, w_ref, bb_ref, out_ref, c_ref):
    g = pl.program_id(0)

    @pl.when(g == 0)
    def _init():
        x = x_ref[...]
        cw = _topk2_coeffs(jnp.dot(x, rw_ref[...], preferred_element_type=jnp.float32))
        cb = _topk2_coeffs(jnp.dot(x, brw_ref[...], preferred_element_type=jnp.float32))
        c_ref[...] = cw
        out_ref[...] = jnp.dot(cb, bb_ref[...], preferred_element_type=jnp.float32)

    c = c_ref[...]
    iota = jax.lax.broadcasted_iota(jnp.int32, c.shape, 1)
    acc = out_ref[...]
    for j in range(_EB):
        e = g * _EB + j
        y = jnp.dot(x_ref[...], w_ref[j], preferred_element_type=jnp.float32)
        ce = jnp.sum(jnp.where(iota == e, c, 0.0), axis=1, keepdims=True)
        acc = acc + ce * y
    out_ref[...] = acc


def kernel(input_batch, router_w, bias_router_w, weight_bank, bias_bank):
    return pl.pallas_call(
        _moe_kernel,
        grid=(_E // _EB,),
        in_specs=[
            pl.BlockSpec((_B, _D), lambda e: (0, 0)),
            pl.BlockSpec((_D, _E), lambda e: (0, 0)),
            pl.BlockSpec((_D, _E), lambda e: (0, 0)),
            pl.BlockSpec((_EB, _D, _O), lambda e: (e, 0, 0)),
            pl.BlockSpec((_E, _O), lambda e: (0, 0)),
        ],
        out_specs=pl.BlockSpec((_B, _O), lambda e: (0, 0)),
        out_shape=jax.ShapeDtypeStruct((_B, _O), jnp.float32),
        scratch_shapes=[pltpu.VMEM((_B, _E), jnp.float32)],
        compiler_params=pltpu.CompilerParams(
            dimension_semantics=("arbitrary",),
        ),
    )(input_batch, router_w, bias_router_w, weight_bank, bias_bank)


# speedup vs baseline: 55.9181x; 1.0848x over previous
import jax
import jax.numpy as jnp
from jax.experimental import pallas as pl
from jax.experimental.pallas import tpu as pltpu

_B, _D, _O, _E = 256, 768, 256, 16


def _topk2_coeffs(logits):
    iota = jax.lax.broadcasted_iota(jnp.int32, logits.shape, 1)
    l1 = jnp.max(logits, axis=-1, keepdims=True)
    i1 = jnp.min(jnp.where(logits == l1, iota, _E), axis=-1, keepdims=True)
    masked = jnp.where(iota == i1, -jnp.inf, logits)
    l2 = jnp.max(masked, axis=-1, keepdims=True)
    i2 = jnp.min(jnp.where(masked == l2, iota, _E), axis=-1, keepdims=True)
    p1 = 1.0 / (1.0 + jnp.exp(l2 - l1))
    return jnp.where(iota == i1, p1, 0.0) + jnp.where(iota == i2, 1.0 - p1, 0.0)


_EB = 8


def _moe_kernel(x_ref, rw_ref, brw_ref, w_ref, bb_ref, out_ref, c_ref):
    g = pl.program_id(0)

    @pl.when(g == 0)
    def _init():
        x = x_ref[...]
        cw = _topk2_coeffs(jnp.dot(x, rw_ref[...], preferred_element_type=jnp.float32))
        cb = _topk2_coeffs(jnp.dot(x, brw_ref[...], preferred_element_type=jnp.float32))
        c_ref[...] = cw
        out_ref[...] = jnp.dot(cb, bb_ref[...], preferred_element_type=jnp.float32)

    c = c_ref[...]
    iota = jax.lax.broadcasted_iota(jnp.int32, c.shape, 1)
    acc = out_ref[...]
    for j in range(_EB):
        e = g * _EB + j
        y = jnp.dot(x_ref[...], w_ref[j], preferred_element_type=jnp.float32)
        ce = jnp.sum(jnp.where(iota == e, c, 0.0), axis=1, keepdims=True)
        acc = acc + ce * y
    out_ref[...] = acc


def kernel(input_batch, router_w, bias_router_w, weight_bank, bias_bank):
    return pl.pallas_call(
        _moe_kernel,
        grid=(_E // _EB,),
        in_specs=[
            pl.BlockSpec((_B, _D), lambda e: (0, 0)),
            pl.BlockSpec((_D, _E), lambda e: (0, 0)),
            pl.BlockSpec((_D, _E), lambda e: (0, 0)),
            pl.BlockSpec((_EB, _D, _O), lambda e: (e, 0, 0)),
            pl.BlockSpec((_E, _O), lambda e: (0, 0)),
        ],
        out_specs=pl.BlockSpec((_B, _O), lambda e: (0, 0)),
        out_shape=jax.ShapeDtypeStruct((_B, _O), jnp.float32),
        scratch_shapes=[pltpu.VMEM((_B, _E), jnp.float32)],
        compiler_params=pltpu.CompilerParams(
            dimension_semantics=("arbitrary",),
        ),
    )(input_batch, router_w, bias_router_w, weight_bank, bias_bank)
